# Initial kernel scaffold; baseline (speedup 1.0000x reference)
import jax
import jax.numpy as jnp
from jax.experimental import pallas as pl


def kernel(x, iy, acq_params, emb, W1, b1, W2, b2, std):
    raise NotImplementedError("write your pallas kernel here")


# speedup vs baseline: 6.6804x; 6.6804x over previous
import functools

import jax
import jax.numpy as jnp
from jax import lax
from jax.experimental import pallas as pl
from jax.experimental.pallas import tpu as pltpu
from jax.experimental.pallas import tpu_sc as plsc

B = 4096
L = 64
NUM_COLS = 736
BLOCK_B = 256

_NC = 2
_NS = 16
_NW = _NC * _NS
_ROWS_PER_W = B // _NW
_GROUPS = _ROWS_PER_W // 16


def _sc_body(iy_hbm, acq_hbm, emb_hbm, consts_hbm, out_hbm,
             iy_v, acq_v, emb_v, consts_v, iys_v):
    wid = lax.axis_index("s") * _NC + lax.axis_index("c")
    base = wid * _ROWS_PER_W

    pltpu.sync_copy(iy_hbm.at[pl.ds(base, _ROWS_PER_W)], iy_v)
    pltpu.sync_copy(acq_hbm.at[pl.ds(base, _ROWS_PER_W)], acq_v)
    pltpu.sync_copy(emb_hbm, emb_v)
    pltpu.sync_copy(consts_hbm, consts_v)

    w10 = consts_v[0]
    w11 = consts_v[1]
    b10 = consts_v[2]
    b11 = consts_v[3]
    w200 = consts_v[4]
    w201 = consts_v[5]
    w210 = consts_v[6]
    w211 = consts_v[7]
    b20 = consts_v[8]
    b21 = consts_v[9]

    lane = lax.iota(jnp.int32, 16)

    def group(g, _):
        iy16 = iy_v[pl.ds(g * 16, 16)]
        acq16 = acq_v[pl.ds(g * 16, 16)]
        h0 = jnp.maximum(acq16 * w10 + b10, 0.0)
        h1 = jnp.maximum(acq16 * w11 + b11, 0.0)
        a = (h0 * w200 + h1 * w201) + b20
        c = (h0 * w210 + h1 * w211) + b21
        rows = g * 16 + lane
        for l in range(L):
            v = plsc.load_gather(emb_v, [iy16 + l])
            t = jnp.maximum(v * a + c, 0.0) + 1e-6
            plsc.store_scatter(iys_v, [rows, lane * 0 + l], t)
        return 0

    lax.fori_loop(0, _GROUPS, group, 0, unroll=False)
    pltpu.sync_copy(iys_v, out_hbm.at[pl.ds(base, _ROWS_PER_W), :])


@functools.partial(
    pl.kernel,
    mesh=plsc.VectorSubcoreMesh(core_axis_name="c", subcore_axis_name="s"),
    out_type=jax.ShapeDtypeStruct((B, L), jnp.float32),
    compiler_params=pltpu.CompilerParams(needs_layout_passes=False),
    scratch_types=[
        pltpu.VMEM((_ROWS_PER_W,), jnp.int32),
        pltpu.VMEM((_ROWS_PER_W,), jnp.float32),
        pltpu.VMEM((NUM_COLS,), jnp.float32),
        pltpu.VMEM((10, 16), jnp.float32),
        pltpu.VMEM((_ROWS_PER_W, L), jnp.float32),
    ],
)
def _sc_rows(iy_hbm, acq_hbm, emb_hbm, consts_hbm, out_hbm,
             iy_v, acq_v, emb_v, consts_v, iys_v):
    _sc_body(iy_hbm, acq_hbm, emb_hbm, consts_hbm, out_hbm,
             iy_v, acq_v, emb_v, consts_v, iys_v)


def _tc_body(x_ref, iys_ref, std_ref, out_ref):
    iys = iys_ref[...]
    inv = 1.0 / iys
    inv = inv * (2.0 - iys * inv)
    inv = inv * (2.0 - iys * inv)
    sv = std_ref[0:1, 0:1] * inv
    ivar = sv * sv
    xv = jnp.clip(x_ref[...], 1e-6, 1e9)
    out_ref[...] = jnp.sqrt(xv * inv[:, None, :] + ivar[:, None, :])


@jax.jit
def kernel(x, iy, acq_params, emb, W1, b1, W2, b2, std):
    x3 = x.reshape(B, 64, L)
    consts = jnp.tile(
        jnp.stack([W1[0, 0], W1[1, 0], b1[0], b1[1],
                   W2[0, 0], W2[0, 1], W2[1, 0], W2[1, 1],
                   b2[0], b2[1]])[:, None], (1, 16))
    iys = _sc_rows(iy.astype(jnp.int32), acq_params[:, 0], emb[:, 0], consts)

    nb = B // BLOCK_B
    out = pl.pallas_call(
        _tc_body,
        grid=(nb,),
        in_specs=[
            pl.BlockSpec((BLOCK_B, 64, L), lambda i: (i, 0, 0)),
            pl.BlockSpec((BLOCK_B, L), lambda i: (i, 0)),
            pl.BlockSpec((1, 1), lambda i: (0, 0)),
        ],
        out_specs=pl.BlockSpec((BLOCK_B, 64, L), lambda i: (i, 0, 0)),
        out_shape=jax.ShapeDtypeStruct((B, 64, L), jnp.float32),
    )(x3, iys, std)
    return out.reshape(B, 8, 8, L)
